# baseline (speedup 1.0000x reference)
import jax
import jax.numpy as jnp
from jax import lax
from jax.experimental import pallas as pl
from jax.experimental.pallas import tpu as pltpu
from jax.experimental.pallas import tpu_sc as plsc

_VOCAB = 1000000
_EMBED_DIM = 32
_PAD_DIM = 128
_BATCH = 16384
_FIELDS = 100

_N = _BATCH * _FIELDS
_NW = 32
_UNIT = 256
_NBB = _UNIT // 128
_SB = _BATCH // _UNIT
_U_TOTAL = _FIELDS * _SB
_U_PER_W = _U_TOTAL // _NW


def _gather_body(table_hbm, idx_hbm, out_hbm,
                 idx0, idx1, rows0, rows1, tr0, tr1,
                 i0, i1, g0, g1, s0, s1):
    c = lax.axis_index("c")
    s = lax.axis_index("s")
    wid = s * 2 + c
    ubase = wid * _U_PER_W

    idx = (idx0, idx1)
    rows = (rows0, rows1)
    tr = (tr0, tr1)
    isem = (i0, i1)
    gs = (g0, g1)
    ss = (s0, s1)

    io = lax.iota(jnp.int32, 16)
    e_lo = io
    e_hi = io + 16

    def start_idx(lu, b):
        lu = jnp.minimum(lu, _U_PER_W - 1)
        src = idx_hbm.at[pl.ds((ubase + lu) * _UNIT, _UNIT)]
        pltpu.async_copy(src, idx[b], isem[b])

    def wait_idx(b):
        pltpu.make_async_copy(idx_hbm.at[pl.ds(0, _UNIT)], idx[b], isem[b]).wait()

    def start_gather(b):
        pltpu.async_copy(table_hbm.at[idx[b]], rows[b], gs[b])

    def wait_gather(b):
        pltpu.make_async_copy(table_hbm.at[pl.ds(0, _UNIT), :], rows[b], gs[b]).wait()

    def transpose_unit(b):
        rb = rows[b]
        tb = tr[b]
        for bb in range(_NBB):
            bbv = jnp.full((16,), bb, jnp.int32)
            base = bb * 128

            def body16(k, carry, bbv=bbv, base=base):
                c0 = k * 16
                for kk in range(16):
                    ci = c0 + kk
                    j = base + ci
                    v0 = rb[j, pl.ds(0, 16)]
                    v1 = rb[j, pl.ds(16, 16)]
                    colv = jnp.full((16,), 0, jnp.int32) + ci
                    plsc.store_scatter(tb, [bbv, e_lo, colv], v0)
                    plsc.store_scatter(tb, [bbv, e_hi, colv], v1)
                return carry

            lax.fori_loop(0, 8, body16, 0)

    def start_store(lu, b):
        u = ubase + lu
        f = u // _SB
        sb = u % _SB
        for t in range(4):
            for bb in range(_NBB):
                src = tr[b].at[bb, pl.ds(t * 8, 8), pl.ds(0, 128)]
                dst = out_hbm.at[f, pl.ds(t * 1024 + sb * (8 * _NBB) + bb * 8, 8), :]
                pltpu.async_copy(src, dst, ss[b])

    def wait_store(b):
        pltpu.make_async_copy(
            rows[b].at[pl.ds(0, 64), :], out_hbm.at[0, pl.ds(0, 64), :],
            ss[b]).wait()

    def step(lu, b, first):
        o = 1 - b
        wait_gather(b)
        start_idx(lu + 2, b)
        wait_idx(o)
        start_gather(o)
        if not first:
            wait_store(b)
        transpose_unit(b)
        start_store(lu, b)

    start_idx(0, 0)
    wait_idx(0)
    start_gather(0)
    start_idx(1, 1)
    step(0, 0, True)
    step(1, 1, True)

    def pair(g, carry):
        step(2 * g, 0, False)
        step(2 * g + 1, 1, False)
        return carry

    lax.fori_loop(1, _U_PER_W // 2, pair, 0)

    wait_gather(0)
    wait_idx(1)
    wait_store(0)
    wait_store(1)


@jax.jit
def _gather(table, idx_flat):
    mesh = plsc.VectorSubcoreMesh(core_axis_name="c", subcore_axis_name="s")
    fn = pl.kernel(
        _gather_body,
        mesh=mesh,
        out_type=jax.ShapeDtypeStruct((_FIELDS, 4096, 128), jnp.float32),
        scratch_types=[
            pltpu.VMEM((_UNIT,), jnp.int32),
            pltpu.VMEM((_UNIT,), jnp.int32),
            pltpu.VMEM((_UNIT, _PAD_DIM), jnp.float32),
            pltpu.VMEM((_UNIT, _PAD_DIM), jnp.float32),
            pltpu.VMEM((_NBB, 32, 129), jnp.float32),
            pltpu.VMEM((_NBB, 32, 129), jnp.float32),
            pltpu.SemaphoreType.DMA,
            pltpu.SemaphoreType.DMA,
            pltpu.SemaphoreType.DMA,
            pltpu.SemaphoreType.DMA,
            pltpu.SemaphoreType.DMA,
            pltpu.SemaphoreType.DMA,
        ],
        compiler_params=pltpu.CompilerParams(
            use_tc_tiling_on_sc=False, needs_layout_passes=False),
    )
    return fn(table, idx_flat)


def kernel(x, class_embedding):
    table128 = jnp.pad(class_embedding, ((0, 0), (0, _PAD_DIM - _EMBED_DIM)))
    idx_flat = x.astype(jnp.int32).T.reshape(-1)
    out3 = _gather(table128, idx_flat)
    out5 = out3.reshape(_FIELDS, 4, 128, 8, 128)
    return out5.transpose(2, 4, 0, 1, 3).reshape(_BATCH, _FIELDS, _EMBED_DIM)


# speedup vs baseline: 1.1688x; 1.1688x over previous
import jax
import jax.numpy as jnp
from jax import lax
from jax.experimental import pallas as pl
from jax.experimental.pallas import tpu as pltpu
from jax.experimental.pallas import tpu_sc as plsc

_VOCAB = 1000000
_EMBED_DIM = 32
_BATCH = 16384
_FIELDS = 100

_N = _BATCH * _FIELDS
_NW = 32
_UNIT = 512
_SB = _BATCH // _UNIT
_U_TOTAL = _FIELDS * _SB
_U_PER_W = _U_TOTAL // _NW
_B_PER_W = _U_PER_W * _UNIT


def _gather_body(table_hbm, idx_hbm, out_hbm,
                 idx_all, rows0, rows1, tr0, tr1, g0, g1, s0, s1):
    c = lax.axis_index("c")
    s = lax.axis_index("s")
    wid = s * 2 + c
    ubase = wid * _U_PER_W

    pltpu.sync_copy(idx_hbm.at[pl.ds(wid * _B_PER_W, _B_PER_W)], idx_all)

    rows = (rows0, rows1)
    tr = (tr0, tr1)
    gs = (g0, g1)
    ss = (s0, s1)

    io = lax.iota(jnp.int32, 16)
    e_lo = io
    e_hi = io + 16

    def start_gather(lu, b):
        lu = jnp.minimum(lu, _U_PER_W - 1)
        idx_slice = idx_all.at[pl.ds(lu * _UNIT, _UNIT)]
        pltpu.async_copy(table_hbm.at[idx_slice], rows[b], gs[b])

    def wait_gather(b):
        pltpu.make_async_copy(table_hbm.at[pl.ds(0, _UNIT), :], rows[b], gs[b]).wait()

    def transpose_unit(b):
        rb = rows[b]
        tb = tr[b]
        for bb in range(4):
            tbb = tb.at[bb]
            base = bb * 128

            def body16(k, carry, tbb=tbb, base=base):
                c0 = k * 16
                for kk in range(16):
                    ci = c0 + kk
                    j = base + ci
                    v0 = rb[j, pl.ds(0, 16)]
                    v1 = rb[j, pl.ds(16, 16)]
                    colv = jnp.full((16,), 0, jnp.int32) + ci
                    plsc.store_scatter(tbb, [e_lo, colv], v0)
                    plsc.store_scatter(tbb, [e_hi, colv], v1)
                return carry

            lax.fori_loop(0, 8, body16, 0)

    def start_store(lu, b):
        u = ubase + lu
        f = u // _SB
        sb = u % _SB
        for t in range(4):
            for bb in range(4):
                src = tr[b].at[bb, pl.ds(t * 8, 8), pl.ds(0, 128)]
                dst = out_hbm.at[f, pl.ds(t * 1024 + sb * 32 + bb * 8, 8), :]
                pltpu.async_copy(src, dst, ss[b])

    def wait_store(b):
        pltpu.make_async_copy(
            rows[b], out_hbm.at[0, pl.ds(0, _UNIT), pl.ds(0, _EMBED_DIM)],
            ss[b]).wait()

    start_gather(0, 0)
    wait_gather(0)
    start_gather(1, 1)
    transpose_unit(0)
    start_store(0, 0)
    wait_gather(1)
    start_gather(2, 0)
    transpose_unit(1)
    start_store(1, 1)

    def pair(g, carry):
        lu0 = 2 * g
        wait_gather(0)
        start_gather(lu0 + 1, 1)
        wait_store(0)
        transpose_unit(0)
        start_store(lu0, 0)
        wait_gather(1)
        start_gather(lu0 + 2, 0)
        wait_store(1)
        transpose_unit(1)
        start_store(lu0 + 1, 1)
        return carry

    lax.fori_loop(1, _U_PER_W // 2, pair, 0)

    wait_gather(0)
    wait_store(0)
    wait_store(1)


@jax.jit
def _gather(table, idx_flat):
    mesh = plsc.VectorSubcoreMesh(core_axis_name="c", subcore_axis_name="s")
    fn = pl.kernel(
        _gather_body,
        mesh=mesh,
        out_type=jax.ShapeDtypeStruct((_FIELDS, 4096, 128), jnp.float32),
        scratch_types=[
            pltpu.VMEM((_B_PER_W,), jnp.int32),
            pltpu.VMEM((_UNIT, _EMBED_DIM), jnp.float32),
            pltpu.VMEM((_UNIT, _EMBED_DIM), jnp.float32),
            pltpu.VMEM((4, 32, 129), jnp.float32),
            pltpu.VMEM((4, 32, 129), jnp.float32),
            pltpu.SemaphoreType.DMA,
            pltpu.SemaphoreType.DMA,
            pltpu.SemaphoreType.DMA,
            pltpu.SemaphoreType.DMA,
        ],
        compiler_params=pltpu.CompilerParams(
            use_tc_tiling_on_sc=False, needs_layout_passes=False),
    )
    return fn(table, idx_flat)


def kernel(x, class_embedding):
    idx_flat = x.astype(jnp.int32).T.reshape(-1)
    out3 = _gather(class_embedding, idx_flat)
    out5 = out3.reshape(_FIELDS, 4, 128, 8, 128)
    return out5.transpose(2, 4, 0, 1, 3).reshape(_BATCH, _FIELDS, _EMBED_DIM)
